# baseline (speedup 1.0000x reference)
import functools

import jax
import jax.numpy as jnp
from jax import lax
from jax.experimental import pallas as pl
from jax.experimental.pallas import tpu as pltpu
from jax.experimental.pallas import tpu_sc as plsc

_B = 320000
_D = 128
_NC = 2
_NS = 16
_NW = _NC * _NS
_BPW = _B // _NW
_C = 40
_NCHUNK = _BPW // _C
_NBUF = 3
_NIB = 2 * _NBUF
_NFULL = _NCHUNK // _NIB


def _make_sc_kernel():
    mesh = plsc.VectorSubcoreMesh(core_axis_name="c", subcore_axis_name="s")

    @functools.partial(
        pl.kernel,
        mesh=mesh,
        out_type=jax.ShapeDtypeStruct((_B, _D), jnp.float32),
        scratch_types=[
            pltpu.VMEM((_NIB, 2 * _C), jnp.int32),
            pltpu.VMEM((_NBUF, 2 * _C, _D // 2), jnp.int32),
            pltpu.VMEM((_NBUF, _C, _D), jnp.float32),
            pltpu.VMEM_SHARED((10000, _D // 2), jnp.int32),
        ]
        + [pltpu.SemaphoreType.DMA] * (_NIB + 2 * _NBUF),
    )
    def sc_kernel(h_hbm, idx_hbm, out_hbm, idx_v, gab, go, h_sh, *sems):
        isem = sems[:_NIB]
        gsem = sems[_NIB:_NIB + _NBUF]
        ssem = sems[_NIB + _NBUF:]
        wid = lax.axis_index("s") * _NC + lax.axis_index("c")
        base = wid * _BPW

        def idx_load(c, k):
            off = 2 * (base + c * _C)
            pltpu.async_copy(idx_hbm.at[pl.ds(off, 2 * _C)], idx_v.at[k],
                             isem[k])

        def idx_wait(c, k):
            off = 2 * (base + c * _C)
            pltpu.make_async_copy(idx_hbm.at[pl.ds(off, 2 * _C)],
                                  idx_v.at[k], isem[k]).wait()

        def gather_issue(k, b):
            pltpu.async_copy(h_sh.at[idx_v.at[k]], gab.at[b], gsem[b])

        def gather_wait(k, b):
            pltpu.make_async_copy(h_sh.at[idx_v.at[k]], gab.at[b],
                                  gsem[b]).wait()

        def store_issue(c, b):
            pltpu.async_copy(
                go.at[b], out_hbm.at[pl.ds(base + c * _C, _C)], ssem[b])

        def store_wait(c, b):
            pltpu.make_async_copy(
                go.at[b], out_hbm.at[pl.ds(base + c * _C, _C)],
                ssem[b]).wait()

        def widen_lo(v):
            sh = jnp.full((16,), 16, dtype=jnp.int32)
            return lax.bitcast_convert_type(
                lax.shift_left(v, sh), jnp.float32)

        def widen_hi(v):
            msk = jnp.full((16,), -65536, dtype=jnp.int32)
            return lax.bitcast_convert_type(
                lax.bitwise_and(v, msk), jnp.float32)

        def compute(b):
            def row(r, u):
                for g in range(_D // 32):
                    sl = pl.ds(g * 16, 16)
                    ai = gab[b, 2 * r, sl]
                    bi = gab[b, 2 * r + 1, sl]
                    go[b, r, pl.ds(g * 32, 16)] = (
                        widen_lo(ai) * widen_lo(bi))
                    go[b, r, pl.ds(g * 32 + 16, 16)] = (
                        widen_hi(ai) * widen_hi(bi))
                return u
            lax.fori_loop(0, _C, row, 0)

        @pl.when(lax.axis_index("s") == 0)
        def _():
            pltpu.sync_copy(h_hbm, h_sh)

        for c in range(_NIB):
            idx_load(c, c)
        for c in range(_NBUF):
            idx_wait(c, c)
        plsc.subcore_barrier()
        for c in range(_NBUF):
            gather_issue(c, c)

        def step(c, k, b, with_store_wait, with_idx_load, prefetch):
            gather_wait(k, b)
            if with_store_wait:
                store_wait(c - _NBUF, b)
            compute(b)
            store_issue(c, b)
            if with_idx_load:
                idx_load(c + _NIB, k)
            if prefetch:
                idx_wait(c + _NBUF, (k + _NBUF) % _NIB)
                gather_issue((k + _NBUF) % _NIB, b)

        for c in range(_NIB):
            step(c, c, c % _NBUF, c >= _NBUF, True, True)

        def dgroup(g, u):
            c0 = g * _NIB
            for k in range(_NIB):
                step(c0 + k, k, k % _NBUF, True, True, True)
            return u

        lax.fori_loop(1, _NFULL - 1, dgroup, 0)

        for c in range((_NFULL - 1) * _NIB, _NFULL * _NIB):
            step(c, c % _NIB, c % _NBUF, True, c + _NIB < _NCHUNK, True)

        for c in range(_NFULL * _NIB, _NCHUNK):
            k, b = c % _NIB, c % _NBUF
            gather_wait(k, b)
            store_wait(c - _NBUF, b)
            compute(b)
            store_issue(c, b)
            if c + _NBUF < _NCHUNK:
                idx_wait(c + _NBUF, (k + _NBUF) % _NIB)
                gather_issue((k + _NBUF) % _NIB, b)

        for c in range(_NCHUNK - _NBUF, _NCHUNK):
            store_wait(c, c % _NBUF)

    return sc_kernel


_SC_KERNEL = _make_sc_kernel()


def kernel(h, edge_label_index):
    eli_flat = edge_label_index.astype(jnp.int32).T.reshape(2 * _B)
    h_bf = h.reshape(10000, 4, 2, 16).transpose(0, 1, 3, 2)
    h_w = jax.lax.bitcast_convert_type(
        h_bf.astype(jnp.bfloat16), jnp.int32).reshape(10000, 64)
    return _SC_KERNEL(h_w, eli_flat)


# speedup vs baseline: 3.6488x; 3.6488x over previous
import functools

import jax
import jax.numpy as jnp
from jax import lax
from jax.experimental import pallas as pl
from jax.experimental.pallas import tpu as pltpu
from jax.experimental.pallas import tpu_sc as plsc

_B = 320000
_D = 128
_NC = 2
_NS = 16
_NW = _NC * _NS
_BPW = _B // _NW
_C = 40
_NCHUNK = _BPW // _C
_NBUF = 3
_NIB = 2 * _NBUF
_NFULL = _NCHUNK // _NIB


def _make_sc_kernel():
    mesh = plsc.VectorSubcoreMesh(core_axis_name="c", subcore_axis_name="s")

    @functools.partial(
        pl.kernel,
        mesh=mesh,
        out_type=jax.ShapeDtypeStruct((_B, _D), jnp.float32),
        scratch_types=[
            pltpu.VMEM((_NIB, 2 * _C), jnp.int32),
            pltpu.VMEM((_NBUF, 2 * _C, _D // 2), jnp.int32),
            pltpu.VMEM((_NBUF, _C, _D), jnp.float32),
            pltpu.VMEM_SHARED((10000, _D // 2), jnp.int32),
        ]
        + [pltpu.SemaphoreType.DMA] * (_NIB + 2 * _NBUF),
    )
    def sc_kernel(h_hbm, src_hbm, dst_hbm, out_hbm, idx_v, gab, go, h_sh,
                  *sems):
        isem = sems[:_NIB]
        gsem = sems[_NIB:_NIB + _NBUF]
        ssem = sems[_NIB + _NBUF:]
        wid = lax.axis_index("s") * _NC + lax.axis_index("c")
        base = wid * _BPW

        def idx_load(c, k):
            off = base + c * _C
            pltpu.async_copy(src_hbm.at[pl.ds(off, _C)],
                             idx_v.at[k, pl.ds(0, _C)], isem[k])
            pltpu.async_copy(dst_hbm.at[pl.ds(off, _C)],
                             idx_v.at[k, pl.ds(_C, _C)], isem[k])

        def idx_wait(c, k):
            off = base + c * _C
            pltpu.make_async_copy(src_hbm.at[pl.ds(off, _C)],
                                  idx_v.at[k, pl.ds(0, _C)], isem[k]).wait()
            pltpu.make_async_copy(dst_hbm.at[pl.ds(off, _C)],
                                  idx_v.at[k, pl.ds(_C, _C)],
                                  isem[k]).wait()

        def gather_issue(k, b):
            pltpu.async_copy(h_sh.at[idx_v.at[k]], gab.at[b], gsem[b])

        def gather_wait(k, b):
            pltpu.make_async_copy(h_sh.at[idx_v.at[k]], gab.at[b],
                                  gsem[b]).wait()

        def store_issue(c, b):
            pltpu.async_copy(
                go.at[b], out_hbm.at[pl.ds(base + c * _C, _C)], ssem[b])

        def store_wait(c, b):
            pltpu.make_async_copy(
                go.at[b], out_hbm.at[pl.ds(base + c * _C, _C)],
                ssem[b]).wait()

        def widen_lo(v):
            sh = jnp.full((16,), 16, dtype=jnp.int32)
            return lax.bitcast_convert_type(
                lax.shift_left(v, sh), jnp.float32)

        def widen_hi(v):
            msk = jnp.full((16,), -65536, dtype=jnp.int32)
            return lax.bitcast_convert_type(
                lax.bitwise_and(v, msk), jnp.float32)

        def compute(b):
            def row(r, u):
                for g in range(_D // 32):
                    sl = pl.ds(g * 16, 16)
                    ai = gab[b, r, sl]
                    bi = gab[b, _C + r, sl]
                    go[b, r, pl.ds(g * 32, 16)] = (
                        widen_lo(ai) * widen_lo(bi))
                    go[b, r, pl.ds(g * 32 + 16, 16)] = (
                        widen_hi(ai) * widen_hi(bi))
                return u
            lax.fori_loop(0, _C, row, 0)

        @pl.when(lax.axis_index("s") == 0)
        def _():
            pltpu.sync_copy(h_hbm, h_sh)

        for c in range(_NIB):
            idx_load(c, c)
        for c in range(_NBUF):
            idx_wait(c, c)
        plsc.subcore_barrier()
        for c in range(_NBUF):
            gather_issue(c, c)

        def step(c, k, b, with_store_wait, with_idx_load, prefetch):
            gather_wait(k, b)
            if with_store_wait:
                store_wait(c - _NBUF, b)
            compute(b)
            store_issue(c, b)
            if with_idx_load:
                idx_load(c + _NIB, k)
            if prefetch:
                idx_wait(c + _NBUF, (k + _NBUF) % _NIB)
                gather_issue((k + _NBUF) % _NIB, b)

        for c in range(_NIB):
            step(c, c, c % _NBUF, c >= _NBUF, True, True)

        def dgroup(g, u):
            c0 = g * _NIB
            for k in range(_NIB):
                step(c0 + k, k, k % _NBUF, True, True, True)
            return u

        lax.fori_loop(1, _NFULL - 1, dgroup, 0)

        for c in range((_NFULL - 1) * _NIB, _NFULL * _NIB):
            step(c, c % _NIB, c % _NBUF, True, c + _NIB < _NCHUNK, True)

        for c in range(_NFULL * _NIB, _NCHUNK):
            k, b = c % _NIB, c % _NBUF
            gather_wait(k, b)
            store_wait(c - _NBUF, b)
            compute(b)
            store_issue(c, b)
            if c + _NBUF < _NCHUNK:
                idx_wait(c + _NBUF, (k + _NBUF) % _NIB)
                gather_issue((k + _NBUF) % _NIB, b)

        for c in range(_NCHUNK - _NBUF, _NCHUNK):
            store_wait(c, c % _NBUF)

    return sc_kernel


_SC_KERNEL = _make_sc_kernel()


def kernel(h, edge_label_index):
    eli = edge_label_index.astype(jnp.int32)
    h_bf = h.reshape(10000, 4, 2, 16).transpose(0, 1, 3, 2)
    h_w = jax.lax.bitcast_convert_type(
        h_bf.astype(jnp.bfloat16), jnp.int32).reshape(10000, 64)
    return _SC_KERNEL(h_w, eli[0], eli[1])
